# baseline (speedup 1.0000x reference)
import jax
import jax.numpy as jnp
from jax import lax
from jax.experimental import pallas as pl
from jax.experimental.pallas import tpu as pltpu
from jax.experimental.pallas import tpu_sc as plsc

B = 16384
V = 100000
EMB = 32
HID = 128
N_EFF = 1000
N_OUT = 1000

_NC = 2
_NS = 16
_NW = _NC * _NS
_B_PER_W = B // _NW
_GRP = 128 // EMB


def _gather_one(idx_hbm, tbl_hbm, out_hbm, base, idx_v, grp_v, off_v,
                rows_v, ext_v, sem):
  pltpu.sync_copy(idx_hbm.at[pl.ds(base, _B_PER_W)], idx_v)

  def prep(i, _):
    c = idx_v[pl.ds(i * 16, 16)]
    grp_v[pl.ds(i * 16, 16)] = jnp.bitwise_or(
        lax.shift_left(lax.shift_right_logical(c, 9), 7), jnp.bitwise_and(c, 127))
    off_v[pl.ds(i * 16, 16)] = lax.shift_left(
        jnp.bitwise_and(lax.shift_right_logical(c, 7), 3), 5)
    return 0

  lax.fori_loop(0, _B_PER_W // 16, prep, 0)
  pltpu.async_copy(tbl_hbm.at[grp_v], rows_v, sem).wait()

  def ext(g, _):
    offs = off_v[pl.ds(g * 16, 16)]
    for l in range(16):
      j = g * 16 + l
      off = offs[l]
      ext_v[pl.ds(j * EMB, 16)] = rows_v[j, pl.ds(off, 16)]
      ext_v[pl.ds(j * EMB + 16, 16)] = rows_v[j, pl.ds(off + 16, 16)]
    return 0

  lax.fori_loop(0, _B_PER_W // 16, ext, 0)
  pltpu.sync_copy(ext_v, out_hbm.at[pl.ds(base * EMB, _B_PER_W * EMB)])


def _sc_gather_body(drug_hbm, geno_hbm, demb_hbm, gemb_hbm,
                    outd_hbm, outg_hbm,
                    idx_v, grp_v, off_v, rows_v, ext_v, sem):
  wid = lax.axis_index("s") * _NC + lax.axis_index("c")
  base = wid * _B_PER_W
  _gather_one(drug_hbm, demb_hbm, outd_hbm, base, idx_v, grp_v, off_v,
              rows_v, ext_v, sem)
  _gather_one(geno_hbm, gemb_hbm, outg_hbm, base, idx_v, grp_v, off_v,
              rows_v, ext_v, sem)


_sc_gather = pl.kernel(
    _sc_gather_body,
    out_type=(
        jax.ShapeDtypeStruct((B * EMB,), jnp.float32),
        jax.ShapeDtypeStruct((B * EMB,), jnp.float32),
    ),
    mesh=plsc.VectorSubcoreMesh(core_axis_name="c", subcore_axis_name="s"),
    scratch_types=[
        pltpu.VMEM((_B_PER_W,), jnp.int32),
        pltpu.VMEM((_B_PER_W,), jnp.int32),
        pltpu.VMEM((_B_PER_W,), jnp.int32),
        pltpu.VMEM((_B_PER_W, 128), jnp.float32),
        pltpu.VMEM((_B_PER_W * EMB,), jnp.float32),
        pltpu.SemaphoreType.DMA,
    ],
    compiler_params=pltpu.CompilerParams(use_tc_tiling_on_sc=True),
)


_TB = 512
_TG = 196
_OROWS = _TG * _TB // _GRP


def _tp_body(dT_ref, gT_ref, od_ref, og_ref):
  for src, dst in ((dT_ref, od_ref), (gT_ref, og_ref)):
    x = src[...]
    parts = [jnp.transpose(x[:, 128 * q:128 * (q + 1)]) for q in range(_GRP)]
    dst[...] = jnp.concatenate(parts, axis=1)


def _relayout(dT, gT):
  return pl.pallas_call(
      _tp_body,
      grid=(_TG,),
      in_specs=[
          pl.BlockSpec((EMB, _TB), lambda i: (0, i)),
          pl.BlockSpec((EMB, _TB), lambda i: (0, i)),
      ],
      out_specs=[
          pl.BlockSpec((_TB // _GRP, 128), lambda i: (i, 0)),
          pl.BlockSpec((_TB // _GRP, 128), lambda i: (i, 0)),
      ],
      out_shape=[
          jax.ShapeDtypeStruct((_OROWS, 128), jnp.float32),
          jax.ShapeDtypeStruct((_OROWS, 128), jnp.float32),
      ],
  )(dT, gT)


_BM = 512
_DN = (((0,), (1,)), ((), ()))


def _mlp_body(xd_ref, xg_ref, w1d_ref, w1g_ref, b1_ref, w2_ref, b2_ref,
              we_ref, be_ref, wo_ref, bo_ref, effT_ref, outT_ref):
  h = jnp.dot(xd_ref[...], w1d_ref[...], preferred_element_type=jnp.float32)
  h += jnp.dot(xg_ref[...], w1g_ref[...], preferred_element_type=jnp.float32)
  h = jnp.maximum(h + b1_ref[...], 0.0)
  h = jnp.dot(h, w2_ref[...], preferred_element_type=jnp.float32)
  h = jnp.maximum(h + b2_ref[...], 0.0)
  effT = lax.dot_general(we_ref[...], h, _DN, preferred_element_type=jnp.float32)
  outT = lax.dot_general(wo_ref[...], h, _DN, preferred_element_type=jnp.float32)
  effT_ref[...] = effT + jnp.transpose(be_ref[...])
  outT_ref[...] = outT + jnp.transpose(bo_ref[...])


def _mlp(xd, xg, W1, b1, W2, b2, We, be, Wo, bo):
  w1d = W1[:EMB]
  w1g = W1[EMB:]
  grid = (B // _BM,)
  full = lambda shape: pl.BlockSpec(shape, lambda i: (0, 0))
  return pl.pallas_call(
      _mlp_body,
      grid=grid,
      in_specs=[
          pl.BlockSpec((_BM, EMB), lambda i: (i, 0)),
          pl.BlockSpec((_BM, EMB), lambda i: (i, 0)),
          full((EMB, HID)),
          full((EMB, HID)),
          full((1, HID)),
          full((HID, HID // 2)),
          full((1, HID // 2)),
          full((HID // 2, N_EFF)),
          full((1, N_EFF)),
          full((HID // 2, N_OUT)),
          full((1, N_OUT)),
      ],
      out_specs=[
          pl.BlockSpec((N_EFF, _BM), lambda i: (0, i)),
          pl.BlockSpec((N_OUT, _BM), lambda i: (0, i)),
      ],
      out_shape=[
          jax.ShapeDtypeStruct((N_EFF, B), jnp.float32),
          jax.ShapeDtypeStruct((N_OUT, B), jnp.float32),
      ],
  )(xd, xg, w1d, w1g, b1.reshape(1, HID), W2, b2.reshape(1, HID // 2),
    We, be.reshape(1, N_EFF), Wo, bo.reshape(1, N_OUT))


def kernel(drug, genotype, drug_emb, geno_emb, W1, b1, W2, b2, We, be, Wo, bo):
  dembq, gembq = _relayout(drug_emb.T, geno_emb.T)
  drug_f, geno_f = _sc_gather(drug.astype(jnp.int32), genotype.astype(jnp.int32),
                              dembq, gembq)
  effT, outT = _mlp(drug_f.reshape(B, EMB), geno_f.reshape(B, EMB),
                    W1, b1, W2, b2, We, be, Wo, bo)
  return (effT.T, outT.T)


# speedup vs baseline: 1.4895x; 1.4895x over previous
import jax
import jax.numpy as jnp
from jax import lax
from jax.experimental import pallas as pl
from jax.experimental.pallas import tpu as pltpu
from jax.experimental.pallas import tpu_sc as plsc

B = 16384
V = 100000
EMB = 32
HID = 128
N_EFF = 1000
N_OUT = 1000

_NC = 2
_NS = 16
_NW = _NC * _NS
_B_PER_W = B // _NW
_GRP = 128 // EMB


def _gather_one(idx_hbm, tbl_hbm, out_hbm, base, idx_v, grp_v, off_v,
                rows_v, ext_v, sem):
  pltpu.sync_copy(idx_hbm.at[pl.ds(base, _B_PER_W)], idx_v)

  def prep(i, _):
    c = idx_v[pl.ds(i * 16, 16)]
    grp_v[pl.ds(i * 16, 16)] = jnp.bitwise_or(
        lax.shift_left(lax.shift_right_logical(c, 9), 7), jnp.bitwise_and(c, 127))
    off_v[pl.ds(i * 16, 16)] = lax.shift_left(
        jnp.bitwise_and(lax.shift_right_logical(c, 7), 3), 5)
    return 0

  lax.fori_loop(0, _B_PER_W // 16, prep, 0)
  pltpu.async_copy(tbl_hbm.at[grp_v], rows_v, sem).wait()

  def ext(g, _):
    offs = off_v[pl.ds(g * 16, 16)]
    for l in range(16):
      j = g * 16 + l
      off = offs[l]
      ext_v[pl.ds(j * EMB, 16)] = rows_v[j, pl.ds(off, 16)]
      ext_v[pl.ds(j * EMB + 16, 16)] = rows_v[j, pl.ds(off + 16, 16)]
    return 0

  lax.fori_loop(0, _B_PER_W // 16, ext, 0)
  pltpu.sync_copy(ext_v, out_hbm.at[pl.ds(base * EMB, _B_PER_W * EMB)])


def _sc_gather_body(drug_hbm, geno_hbm, demb_hbm, gemb_hbm,
                    outd_hbm, outg_hbm,
                    idx_v, grp_v, off_v, rows_v, ext_v, sem):
  wid = lax.axis_index("s") * _NC + lax.axis_index("c")
  base = wid * _B_PER_W
  _gather_one(drug_hbm, demb_hbm, outd_hbm, base, idx_v, grp_v, off_v,
              rows_v, ext_v, sem)
  _gather_one(geno_hbm, gemb_hbm, outg_hbm, base, idx_v, grp_v, off_v,
              rows_v, ext_v, sem)


_sc_gather = pl.kernel(
    _sc_gather_body,
    out_type=(
        jax.ShapeDtypeStruct((B * EMB,), jnp.float32),
        jax.ShapeDtypeStruct((B * EMB,), jnp.float32),
    ),
    mesh=plsc.VectorSubcoreMesh(core_axis_name="c", subcore_axis_name="s"),
    scratch_types=[
        pltpu.VMEM((_B_PER_W,), jnp.int32),
        pltpu.VMEM((_B_PER_W,), jnp.int32),
        pltpu.VMEM((_B_PER_W,), jnp.int32),
        pltpu.VMEM((_B_PER_W, 128), jnp.float32),
        pltpu.VMEM((_B_PER_W * EMB,), jnp.float32),
        pltpu.SemaphoreType.DMA,
    ],
    compiler_params=pltpu.CompilerParams(use_tc_tiling_on_sc=True),
)


_TB = 2048
_TG = 49
_OROWS = _TG * _TB // _GRP
_DT = (((0,), (0,)), ((), ()))


def _tp_body(sel_ref, dT_ref, gT_ref, od_ref, og_ref):
  sel = sel_ref[...]
  for src, dst in ((dT_ref, od_ref), (gT_ref, og_ref)):
    x = src[...]
    for m in range(4):
      acc = None
      for q in range(4):
        xq = x[:, 512 * m + 128 * q:512 * m + 128 * (q + 1)]
        eq = sel[:, 128 * q:128 * (q + 1)]
        t = lax.dot_general(xq, eq, _DT, preferred_element_type=jnp.float32)
        acc = t if acc is None else acc + t
      dst[128 * m:128 * (m + 1), :] = acc


def _relayout(dT, gT):
  cols = jnp.arange(512)
  sel = ((cols[None, :] % 128 - 32 * (cols[None, :] // 128))
         == jnp.arange(EMB)[:, None]).astype(jnp.float32)
  return pl.pallas_call(
      _tp_body,
      grid=(_TG,),
      in_specs=[
          pl.BlockSpec((EMB, 512), lambda i: (0, 0)),
          pl.BlockSpec((EMB, _TB), lambda i: (0, i)),
          pl.BlockSpec((EMB, _TB), lambda i: (0, i)),
      ],
      out_specs=[
          pl.BlockSpec((_TB // _GRP, 128), lambda i: (i, 0)),
          pl.BlockSpec((_TB // _GRP, 128), lambda i: (i, 0)),
      ],
      out_shape=[
          jax.ShapeDtypeStruct((_OROWS, 128), jnp.float32),
          jax.ShapeDtypeStruct((_OROWS, 128), jnp.float32),
      ],
  )(sel, dT, gT)


_BM = 512
_DN = (((0,), (1,)), ((), ()))


def _mlp_body(xd_ref, xg_ref, w1d_ref, w1g_ref, b1_ref, w2_ref, b2_ref,
              we_ref, be_ref, wo_ref, bo_ref, effT_ref, outT_ref):
  h = jnp.dot(xd_ref[...], w1d_ref[...], preferred_element_type=jnp.float32)
  h += jnp.dot(xg_ref[...], w1g_ref[...], preferred_element_type=jnp.float32)
  h = jnp.maximum(h + b1_ref[...], 0.0)
  h = jnp.dot(h, w2_ref[...], preferred_element_type=jnp.float32)
  h = jnp.maximum(h + b2_ref[...], 0.0)
  effT = lax.dot_general(we_ref[...], h, _DN, preferred_element_type=jnp.float32)
  outT = lax.dot_general(wo_ref[...], h, _DN, preferred_element_type=jnp.float32)
  effT_ref[...] = effT + jnp.transpose(be_ref[...])
  outT_ref[...] = outT + jnp.transpose(bo_ref[...])


def _mlp(xd, xg, W1, b1, W2, b2, We, be, Wo, bo):
  w1d = W1[:EMB]
  w1g = W1[EMB:]
  grid = (B // _BM,)
  full = lambda shape: pl.BlockSpec(shape, lambda i: (0, 0))
  return pl.pallas_call(
      _mlp_body,
      grid=grid,
      in_specs=[
          pl.BlockSpec((_BM, EMB), lambda i: (i, 0)),
          pl.BlockSpec((_BM, EMB), lambda i: (i, 0)),
          full((EMB, HID)),
          full((EMB, HID)),
          full((1, HID)),
          full((HID, HID // 2)),
          full((1, HID // 2)),
          full((HID // 2, N_EFF)),
          full((1, N_EFF)),
          full((HID // 2, N_OUT)),
          full((1, N_OUT)),
      ],
      out_specs=[
          pl.BlockSpec((N_EFF, _BM), lambda i: (0, i)),
          pl.BlockSpec((N_OUT, _BM), lambda i: (0, i)),
      ],
      out_shape=[
          jax.ShapeDtypeStruct((N_EFF, B), jnp.float32),
          jax.ShapeDtypeStruct((N_OUT, B), jnp.float32),
      ],
  )(xd, xg, w1d, w1g, b1.reshape(1, HID), W2, b2.reshape(1, HID // 2),
    We, be.reshape(1, N_EFF), Wo, bo.reshape(1, N_OUT))


def kernel(drug, genotype, drug_emb, geno_emb, W1, b1, W2, b2, We, be, Wo, bo):
  dembq, gembq = _relayout(drug_emb.T, geno_emb.T)
  drug_f, geno_f = _sc_gather(drug.astype(jnp.int32), genotype.astype(jnp.int32),
                              dembq, gembq)
  effT, outT = _mlp(drug_f.reshape(B, EMB), geno_f.reshape(B, EMB),
                    W1, b1, W2, b2, We, be, Wo, bo)
  return (effT.T, outT.T)


# speedup vs baseline: 1.5119x; 1.0151x over previous
import jax
import jax.numpy as jnp
from jax import lax
from jax.experimental import pallas as pl
from jax.experimental.pallas import tpu as pltpu
from jax.experimental.pallas import tpu_sc as plsc

B = 16384
V = 100000
EMB = 32
HID = 128
N_EFF = 1000
N_OUT = 1000

_NC = 2
_NS = 16
_NW = _NC * _NS
_CHUNKS = 2
_BC = B // _CHUNKS
_B_PER_W = _BC // _NW
_GRP = 128 // EMB


def _gather_one(idx_hbm, tbl_hbm, out_hbm, base, idx_v, grp_v, off_v,
                rows_v, ext_v, sem):
  pltpu.sync_copy(idx_hbm.at[pl.ds(base, _B_PER_W)], idx_v)

  def prep(i, _):
    c = idx_v[pl.ds(i * 16, 16)]
    grp_v[pl.ds(i * 16, 16)] = jnp.bitwise_or(
        lax.shift_left(lax.shift_right_logical(c, 9), 7), jnp.bitwise_and(c, 127))
    off_v[pl.ds(i * 16, 16)] = lax.shift_left(
        jnp.bitwise_and(lax.shift_right_logical(c, 7), 3), 5)
    return 0

  lax.fori_loop(0, _B_PER_W // 16, prep, 0)
  pltpu.async_copy(tbl_hbm.at[grp_v], rows_v, sem).wait()

  def ext(g, _):
    offs = off_v[pl.ds(g * 16, 16)]
    for l in range(16):
      j = g * 16 + l
      off = offs[l]
      ext_v[pl.ds(j * EMB, 16)] = rows_v[j, pl.ds(off, 16)]
      ext_v[pl.ds(j * EMB + 16, 16)] = rows_v[j, pl.ds(off + 16, 16)]
    return 0

  lax.fori_loop(0, _B_PER_W // 16, ext, 0)
  pltpu.sync_copy(ext_v, out_hbm.at[pl.ds(base * EMB, _B_PER_W * EMB)])


def _sc_gather_body(drug_hbm, geno_hbm, demb_hbm, gemb_hbm,
                    outd_hbm, outg_hbm,
                    idx_v, grp_v, off_v, rows_v, ext_v, sem):
  wid = lax.axis_index("s") * _NC + lax.axis_index("c")
  base = wid * _B_PER_W
  _gather_one(drug_hbm, demb_hbm, outd_hbm, base, idx_v, grp_v, off_v,
              rows_v, ext_v, sem)
  _gather_one(geno_hbm, gemb_hbm, outg_hbm, base, idx_v, grp_v, off_v,
              rows_v, ext_v, sem)


_sc_gather = pl.kernel(
    _sc_gather_body,
    out_type=(
        jax.ShapeDtypeStruct((_BC * EMB,), jnp.float32),
        jax.ShapeDtypeStruct((_BC * EMB,), jnp.float32),
    ),
    mesh=plsc.VectorSubcoreMesh(core_axis_name="c", subcore_axis_name="s"),
    scratch_types=[
        pltpu.VMEM((_B_PER_W,), jnp.int32),
        pltpu.VMEM((_B_PER_W,), jnp.int32),
        pltpu.VMEM((_B_PER_W,), jnp.int32),
        pltpu.VMEM((_B_PER_W, 128), jnp.float32),
        pltpu.VMEM((_B_PER_W * EMB,), jnp.float32),
        pltpu.SemaphoreType.DMA,
    ],
    compiler_params=pltpu.CompilerParams(use_tc_tiling_on_sc=True),
)


_TB = 4096
_TG = 25
_OROWS = _TG * _TB // _GRP
_DT = (((0,), (0,)), ((), ()))


def _tp_body(sel_ref, dT_ref, gT_ref, od_ref, og_ref):
  sel = sel_ref[...]
  for src, dst in ((dT_ref, od_ref), (gT_ref, og_ref)):
    x = src[...]
    for m in range(_TB // 512):
      acc = None
      for q in range(4):
        xq = x[:, 512 * m + 128 * q:512 * m + 128 * (q + 1)]
        eq = sel[:, 128 * q:128 * (q + 1)]
        t = lax.dot_general(xq, eq, _DT, preferred_element_type=jnp.float32)
        acc = t if acc is None else acc + t
      dst[128 * m:128 * (m + 1), :] = acc


def _relayout(dT, gT):
  cols = jnp.arange(512)
  sel = ((cols[None, :] % 128 - 32 * (cols[None, :] // 128))
         == jnp.arange(EMB)[:, None]).astype(jnp.float32)
  return pl.pallas_call(
      _tp_body,
      grid=(_TG,),
      in_specs=[
          pl.BlockSpec((EMB, 512), lambda i: (0, 0)),
          pl.BlockSpec((EMB, _TB), lambda i: (0, i)),
          pl.BlockSpec((EMB, _TB), lambda i: (0, i)),
      ],
      out_specs=[
          pl.BlockSpec((_TB // _GRP, 128), lambda i: (i, 0)),
          pl.BlockSpec((_TB // _GRP, 128), lambda i: (i, 0)),
      ],
      out_shape=[
          jax.ShapeDtypeStruct((_OROWS, 128), jnp.float32),
          jax.ShapeDtypeStruct((_OROWS, 128), jnp.float32),
      ],
  )(sel, dT, gT)


_BM = 512
_DN = (((0,), (1,)), ((), ()))


def _mlp_body(xd_ref, xg_ref, w1d_ref, w1g_ref, b1_ref, w2_ref, b2_ref,
              we_ref, be_ref, wo_ref, bo_ref, effT_ref, outT_ref):
  h = jnp.dot(xd_ref[...], w1d_ref[...], preferred_element_type=jnp.float32)
  h += jnp.dot(xg_ref[...], w1g_ref[...], preferred_element_type=jnp.float32)
  h = jnp.maximum(h + b1_ref[...], 0.0)
  h = jnp.dot(h, w2_ref[...], preferred_element_type=jnp.float32)
  h = jnp.maximum(h + b2_ref[...], 0.0)
  effT = lax.dot_general(we_ref[...], h, _DN, preferred_element_type=jnp.float32)
  outT = lax.dot_general(wo_ref[...], h, _DN, preferred_element_type=jnp.float32)
  effT_ref[...] = effT + jnp.transpose(be_ref[...])
  outT_ref[...] = outT + jnp.transpose(bo_ref[...])


def _mlp(xd, xg, W1, b1, W2, b2, We, be, Wo, bo, c, prev):
  w1d = W1[:EMB]
  w1g = W1[EMB:]
  grid = (_BC // _BM,)
  off = c * (_BC // _BM)
  full = lambda shape: pl.BlockSpec(shape, lambda i: (0, 0))
  ins = [
      pl.BlockSpec((_BM, EMB), lambda i: (i, 0)),
      pl.BlockSpec((_BM, EMB), lambda i: (i, 0)),
      full((EMB, HID)),
      full((EMB, HID)),
      full((1, HID)),
      full((HID, HID // 2)),
      full((1, HID // 2)),
      full((HID // 2, N_EFF)),
      full((1, N_EFF)),
      full((HID // 2, N_OUT)),
      full((1, N_OUT)),
  ]
  args = [xd, xg, w1d, w1g, b1.reshape(1, HID), W2, b2.reshape(1, HID // 2),
          We, be.reshape(1, N_EFF), Wo, bo.reshape(1, N_OUT)]
  aliases = {}
  if prev is not None:
    ins = ins + [pl.BlockSpec(memory_space=pl.ANY),
                 pl.BlockSpec(memory_space=pl.ANY)]
    args = args + [prev[0], prev[1]]
    aliases = {11: 0, 12: 1}
  body = _mlp_body if prev is None else _mlp_body_alias
  return pl.pallas_call(
      body,
      grid=grid,
      in_specs=ins,
      out_specs=[
          pl.BlockSpec((N_EFF, _BM), lambda i: (0, i + off)),
          pl.BlockSpec((N_OUT, _BM), lambda i: (0, i + off)),
      ],
      out_shape=[
          jax.ShapeDtypeStruct((N_EFF, B), jnp.float32),
          jax.ShapeDtypeStruct((N_OUT, B), jnp.float32),
      ],
      input_output_aliases=aliases,
  )(*args)


def _mlp_body_alias(xd_ref, xg_ref, w1d_ref, w1g_ref, b1_ref, w2_ref, b2_ref,
                    we_ref, be_ref, wo_ref, bo_ref, pe_ref, po_ref,
                    effT_ref, outT_ref):
  _mlp_body(xd_ref, xg_ref, w1d_ref, w1g_ref, b1_ref, w2_ref, b2_ref,
            we_ref, be_ref, wo_ref, bo_ref, effT_ref, outT_ref)


def kernel(drug, genotype, drug_emb, geno_emb, W1, b1, W2, b2, We, be, Wo, bo):
  dembq, gembq = _relayout(drug_emb.T, geno_emb.T)
  drug = drug.astype(jnp.int32)
  genotype = genotype.astype(jnp.int32)
  prev = None
  for c in range(_CHUNKS):
    sl = slice(c * _BC, (c + 1) * _BC)
    drug_f, geno_f = _sc_gather(drug[sl], genotype[sl], dembq, gembq)
    prev = _mlp(drug_f.reshape(_BC, EMB), geno_f.reshape(_BC, EMB),
                W1, b1, W2, b2, We, be, Wo, bo, c, prev)
  return (prev[0].T, prev[1].T)


# speedup vs baseline: 1.6005x; 1.0586x over previous
import jax
import jax.numpy as jnp
from jax import lax
from jax.experimental import pallas as pl
from jax.experimental.pallas import tpu as pltpu
from jax.experimental.pallas import tpu_sc as plsc

B = 16384
V = 100000
EMB = 32
HID = 128
N_EFF = 1000
N_OUT = 1000

_NC = 2
_NS = 16
_NW = _NC * _NS
_CHUNKS = 2
_BC = B // _CHUNKS
_B_PER_W = _BC // _NW
_GRP = 128 // EMB


def _gather_one(idx_hbm, tbl_hbm, out_hbm, base, idx_v, grp_v, off_v,
                rows_v, ext_v, sem):
  pltpu.sync_copy(idx_hbm.at[pl.ds(base, _B_PER_W)], idx_v)

  def prep(i, _):
    c = idx_v[pl.ds(i * 16, 16)]
    grp_v[pl.ds(i * 16, 16)] = jnp.bitwise_or(
        lax.shift_left(lax.shift_right_logical(c, 9), 7), jnp.bitwise_and(c, 127))
    off_v[pl.ds(i * 16, 16)] = lax.shift_left(
        jnp.bitwise_and(lax.shift_right_logical(c, 7), 3), 5)
    return 0

  lax.fori_loop(0, _B_PER_W // 16, prep, 0)
  pltpu.async_copy(tbl_hbm.at[grp_v], rows_v, sem).wait()

  def ext(g, _):
    offs = off_v[pl.ds(g * 16, 16)]
    for l in range(16):
      j = g * 16 + l
      off = offs[l]
      ext_v[pl.ds(j * EMB, 16)] = rows_v[j, pl.ds(off, 16)]
      ext_v[pl.ds(j * EMB + 16, 16)] = rows_v[j, pl.ds(off + 16, 16)]
    return 0

  lax.fori_loop(0, _B_PER_W // 16, ext, 0)
  pltpu.sync_copy(ext_v, out_hbm.at[pl.ds(base * EMB, _B_PER_W * EMB)])


def _sc_gather_body(drug_hbm, geno_hbm, demb_hbm, gemb_hbm,
                    outd_hbm, outg_hbm,
                    idx_v, grp_v, off_v, rows_v, ext_v, sem):
  wid = lax.axis_index("s") * _NC + lax.axis_index("c")
  base = wid * _B_PER_W
  _gather_one(drug_hbm, demb_hbm, outd_hbm, base, idx_v, grp_v, off_v,
              rows_v, ext_v, sem)
  _gather_one(geno_hbm, gemb_hbm, outg_hbm, base, idx_v, grp_v, off_v,
              rows_v, ext_v, sem)


_sc_gather = pl.kernel(
    _sc_gather_body,
    out_type=(
        jax.ShapeDtypeStruct((_BC * EMB,), jnp.float32),
        jax.ShapeDtypeStruct((_BC * EMB,), jnp.float32),
    ),
    mesh=plsc.VectorSubcoreMesh(core_axis_name="c", subcore_axis_name="s"),
    scratch_types=[
        pltpu.VMEM((_B_PER_W,), jnp.int32),
        pltpu.VMEM((_B_PER_W,), jnp.int32),
        pltpu.VMEM((_B_PER_W,), jnp.int32),
        pltpu.VMEM((_B_PER_W, 128), jnp.float32),
        pltpu.VMEM((_B_PER_W * EMB,), jnp.float32),
        pltpu.SemaphoreType.DMA,
    ],
    compiler_params=pltpu.CompilerParams(use_tc_tiling_on_sc=True),
    cost_estimate=pl.CostEstimate(
        flops=100_000, bytes_accessed=20_000_000, transcendentals=0),
)


_TB = 4096
_TG = 25
_OROWS = _TG * _TB // _GRP
_DT = (((0,), (0,)), ((), ()))


def _tp_body(sel_ref, dT_ref, gT_ref, od_ref, og_ref):
  sel = sel_ref[...]
  for src, dst in ((dT_ref, od_ref), (gT_ref, og_ref)):
    x = src[...]
    for m in range(_TB // 512):
      acc = None
      for q in range(4):
        xq = x[:, 512 * m + 128 * q:512 * m + 128 * (q + 1)]
        eq = sel[:, 128 * q:128 * (q + 1)]
        t = lax.dot_general(xq, eq, _DT, preferred_element_type=jnp.float32)
        acc = t if acc is None else acc + t
      dst[128 * m:128 * (m + 1), :] = acc


def _relayout(dT, gT):
  cols = jnp.arange(512)
  sel = ((cols[None, :] % 128 - 32 * (cols[None, :] // 128))
         == jnp.arange(EMB)[:, None]).astype(jnp.float32)
  return pl.pallas_call(
      _tp_body,
      grid=(_TG,),
      in_specs=[
          pl.BlockSpec((EMB, 512), lambda i: (0, 0)),
          pl.BlockSpec((EMB, _TB), lambda i: (0, i)),
          pl.BlockSpec((EMB, _TB), lambda i: (0, i)),
      ],
      out_specs=[
          pl.BlockSpec((_TB // _GRP, 128), lambda i: (i, 0)),
          pl.BlockSpec((_TB // _GRP, 128), lambda i: (i, 0)),
      ],
      out_shape=[
          jax.ShapeDtypeStruct((_OROWS, 128), jnp.float32),
          jax.ShapeDtypeStruct((_OROWS, 128), jnp.float32),
      ],
  )(sel, dT, gT)


_BM = 1024
_DN = (((0,), (1,)), ((), ()))


def _mlp_body(xd_ref, xg_ref, w1d_ref, w1g_ref, b1_ref, w2_ref, b2_ref,
              we_ref, be_ref, wo_ref, bo_ref, effT_ref, outT_ref):
  h = jnp.dot(xd_ref[...], w1d_ref[...], preferred_element_type=jnp.float32)
  h += jnp.dot(xg_ref[...], w1g_ref[...], preferred_element_type=jnp.float32)
  h = jnp.maximum(h + b1_ref[...], 0.0)
  h = jnp.dot(h, w2_ref[...], preferred_element_type=jnp.float32)
  h = jnp.maximum(h + b2_ref[...], 0.0)
  effT = lax.dot_general(we_ref[...], h, _DN, preferred_element_type=jnp.float32)
  outT = lax.dot_general(wo_ref[...], h, _DN, preferred_element_type=jnp.float32)
  effT_ref[...] = effT + jnp.transpose(be_ref[...])
  outT_ref[...] = outT + jnp.transpose(bo_ref[...])


def _mlp(xd, xg, W1, b1, W2, b2, We, be, Wo, bo, c, prev):
  w1d = W1[:EMB]
  w1g = W1[EMB:]
  grid = (_BC // _BM,)
  off = c * (_BC // _BM)
  full = lambda shape: pl.BlockSpec(shape, lambda i: (0, 0))
  ins = [
      pl.BlockSpec((_BM, EMB), lambda i: (i, 0)),
      pl.BlockSpec((_BM, EMB), lambda i: (i, 0)),
      full((EMB, HID)),
      full((EMB, HID)),
      full((1, HID)),
      full((HID, HID // 2)),
      full((1, HID // 2)),
      full((HID // 2, N_EFF)),
      full((1, N_EFF)),
      full((HID // 2, N_OUT)),
      full((1, N_OUT)),
  ]
  args = [xd, xg, w1d, w1g, b1.reshape(1, HID), W2, b2.reshape(1, HID // 2),
          We, be.reshape(1, N_EFF), Wo, bo.reshape(1, N_OUT)]
  aliases = {}
  if prev is not None:
    ins = ins + [pl.BlockSpec(memory_space=pl.ANY),
                 pl.BlockSpec(memory_space=pl.ANY)]
    args = args + [prev[0], prev[1]]
    aliases = {11: 0, 12: 1}
  body = _mlp_body if prev is None else _mlp_body_alias
  return pl.pallas_call(
      body,
      grid=grid,
      in_specs=ins,
      out_specs=[
          pl.BlockSpec((N_EFF, _BM), lambda i: (0, i + off)),
          pl.BlockSpec((N_OUT, _BM), lambda i: (0, i + off)),
      ],
      out_shape=[
          jax.ShapeDtypeStruct((N_EFF, B), jnp.float32),
          jax.ShapeDtypeStruct((N_OUT, B), jnp.float32),
      ],
      input_output_aliases=aliases,
  )(*args)


def _mlp_body_alias(xd_ref, xg_ref, w1d_ref, w1g_ref, b1_ref, w2_ref, b2_ref,
                    we_ref, be_ref, wo_ref, bo_ref, pe_ref, po_ref,
                    effT_ref, outT_ref):
  _mlp_body(xd_ref, xg_ref, w1d_ref, w1g_ref, b1_ref, w2_ref, b2_ref,
            we_ref, be_ref, wo_ref, bo_ref, effT_ref, outT_ref)


def kernel(drug, genotype, drug_emb, geno_emb, W1, b1, W2, b2, We, be, Wo, bo):
  dembq, gembq = _relayout(drug_emb.T, geno_emb.T)
  drug = drug.astype(jnp.int32)
  genotype = genotype.astype(jnp.int32)
  prev = None
  for c in range(_CHUNKS):
    sl = slice(c * _BC, (c + 1) * _BC)
    drug_f, geno_f = _sc_gather(drug[sl], genotype[sl], dembq, gembq)
    prev = _mlp(drug_f.reshape(_BC, EMB), geno_f.reshape(_BC, EMB),
                W1, b1, W2, b2, We, be, Wo, bo, c, prev)
  return (prev[0].T, prev[1].T)


# speedup vs baseline: 1.9088x; 1.1926x over previous
import jax
import jax.numpy as jnp
from jax import lax
from jax.experimental import pallas as pl
from jax.experimental.pallas import tpu as pltpu
from jax.experimental.pallas import tpu_sc as plsc

B = 16384
V = 100000
EMB = 32
HID = 128
N_EFF = 1000
N_OUT = 1000

_NC = 2
_NS = 16
_NW = _NC * _NS
_CHUNKS = 2
_BC = B // _CHUNKS
_B_PER_W = _BC // _NW
_GRP = 128 // EMB


def _gather_one(idx_hbm, tbl_hbm, out_hbm, base, idx_v, grp_v, off_v,
                rows_v, ext_v, sem):
  pltpu.sync_copy(idx_hbm.at[pl.ds(base, _B_PER_W)], idx_v)

  def prep(i, _):
    c = idx_v[pl.ds(i * 16, 16)]
    grp_v[pl.ds(i * 16, 16)] = jnp.bitwise_or(
        lax.shift_left(lax.shift_right_logical(c, 9), 7), jnp.bitwise_and(c, 127))
    off_v[pl.ds(i * 16, 16)] = lax.shift_left(
        jnp.bitwise_and(lax.shift_right_logical(c, 7), 3), 5)
    return 0

  lax.fori_loop(0, _B_PER_W // 16, prep, 0)
  pltpu.async_copy(tbl_hbm.at[grp_v], rows_v, sem).wait()

  def ext(g, _):
    offs = off_v[pl.ds(g * 16, 16)]
    for l in range(16):
      j = g * 16 + l
      off = offs[l]
      ext_v[j, pl.ds(0, 16)] = rows_v[j, pl.ds(off, 16)]
      ext_v[j, pl.ds(16, 16)] = rows_v[j, pl.ds(off + 16, 16)]
    return 0

  lax.fori_loop(0, _B_PER_W // 16, ext, 0)
  pltpu.sync_copy(ext_v, out_hbm.at[pl.ds(base, _B_PER_W)])


def _sc_gather_body(drug_hbm, geno_hbm, demb_hbm, gemb_hbm,
                    outd_hbm, outg_hbm,
                    idx_v, grp_v, off_v, rows_v, ext_v, sem):
  wid = lax.axis_index("s") * _NC + lax.axis_index("c")
  base = wid * _B_PER_W
  _gather_one(drug_hbm, demb_hbm, outd_hbm, base, idx_v, grp_v, off_v,
              rows_v, ext_v, sem)
  _gather_one(geno_hbm, gemb_hbm, outg_hbm, base, idx_v, grp_v, off_v,
              rows_v, ext_v, sem)


_sc_gather = pl.kernel(
    _sc_gather_body,
    out_type=(
        jax.ShapeDtypeStruct((_BC, EMB), jnp.float32),
        jax.ShapeDtypeStruct((_BC, EMB), jnp.float32),
    ),
    mesh=plsc.VectorSubcoreMesh(core_axis_name="c", subcore_axis_name="s"),
    scratch_types=[
        pltpu.VMEM((_B_PER_W,), jnp.int32),
        pltpu.VMEM((_B_PER_W,), jnp.int32),
        pltpu.VMEM((_B_PER_W,), jnp.int32),
        pltpu.VMEM((_B_PER_W, 128), jnp.float32),
        pltpu.VMEM((_B_PER_W, EMB), jnp.float32),
        pltpu.SemaphoreType.DMA,
    ],
    compiler_params=pltpu.CompilerParams(use_tc_tiling_on_sc=True),
    cost_estimate=pl.CostEstimate(
        flops=100_000, bytes_accessed=20_000_000, transcendentals=0),
)


_TB = 8192
_TG = 13
_OROWS = _TG * _TB // _GRP
_DT = (((0,), (0,)), ((), ()))


def _tp_body(sel_ref, dT_ref, gT_ref, od_ref, og_ref):
  sel = sel_ref[...]
  for src, dst in ((dT_ref, od_ref), (gT_ref, og_ref)):
    x = src[...]
    for m in range(_TB // 512):
      acc = None
      for q in range(4):
        xq = x[:, 512 * m + 128 * q:512 * m + 128 * (q + 1)]
        eq = sel[:, 128 * q:128 * (q + 1)]
        t = lax.dot_general(xq, eq, _DT, preferred_element_type=jnp.float32)
        acc = t if acc is None else acc + t
      dst[128 * m:128 * (m + 1), :] = acc


def _relayout(dT, gT):
  cols = jnp.arange(512)
  sel = ((cols[None, :] % 128 - 32 * (cols[None, :] // 128))
         == jnp.arange(EMB)[:, None]).astype(jnp.float32)
  return pl.pallas_call(
      _tp_body,
      grid=(_TG,),
      in_specs=[
          pl.BlockSpec((EMB, 512), lambda i: (0, 0)),
          pl.BlockSpec((EMB, _TB), lambda i: (0, i)),
          pl.BlockSpec((EMB, _TB), lambda i: (0, i)),
      ],
      out_specs=[
          pl.BlockSpec((_TB // _GRP, 128), lambda i: (i, 0)),
          pl.BlockSpec((_TB // _GRP, 128), lambda i: (i, 0)),
      ],
      out_shape=[
          jax.ShapeDtypeStruct((_OROWS, 128), jnp.float32),
          jax.ShapeDtypeStruct((_OROWS, 128), jnp.float32),
      ],
  )(sel, dT, gT)


_BM = 1024
_DN = (((0,), (1,)), ((), ()))


def _mlp_body(xd_ref, xg_ref, w1d_ref, w1g_ref, b1_ref, w2_ref, b2_ref,
              we_ref, be_ref, wo_ref, bo_ref, effT_ref, outT_ref):
  h = jnp.dot(xd_ref[...], w1d_ref[...], preferred_element_type=jnp.float32)
  h += jnp.dot(xg_ref[...], w1g_ref[...], preferred_element_type=jnp.float32)
  h = jnp.maximum(h + b1_ref[...], 0.0)
  h = jnp.dot(h, w2_ref[...], preferred_element_type=jnp.float32)
  h = jnp.maximum(h + b2_ref[...], 0.0)
  effT = lax.dot_general(we_ref[...], h, _DN, preferred_element_type=jnp.float32)
  outT = lax.dot_general(wo_ref[...], h, _DN, preferred_element_type=jnp.float32)
  effT_ref[...] = effT + jnp.transpose(be_ref[...])
  outT_ref[...] = outT + jnp.transpose(bo_ref[...])


def _mlp(xd, xg, W1, b1, W2, b2, We, be, Wo, bo, c, prev):
  w1d = W1[:EMB]
  w1g = W1[EMB:]
  grid = (_BC // _BM,)
  off = c * (_BC // _BM)
  full = lambda shape: pl.BlockSpec(shape, lambda i: (0, 0))
  ins = [
      pl.BlockSpec((_BM, EMB), lambda i: (i, 0)),
      pl.BlockSpec((_BM, EMB), lambda i: (i, 0)),
      full((EMB, HID)),
      full((EMB, HID)),
      full((1, HID)),
      full((HID, HID // 2)),
      full((1, HID // 2)),
      full((HID // 2, N_EFF)),
      full((1, N_EFF)),
      full((HID // 2, N_OUT)),
      full((1, N_OUT)),
  ]
  args = [xd, xg, w1d, w1g, b1.reshape(1, HID), W2, b2.reshape(1, HID // 2),
          We, be.reshape(1, N_EFF), Wo, bo.reshape(1, N_OUT)]
  aliases = {}
  if prev is not None:
    ins = ins + [pl.BlockSpec(memory_space=pl.ANY),
                 pl.BlockSpec(memory_space=pl.ANY)]
    args = args + [prev[0], prev[1]]
    aliases = {11: 0, 12: 1}
  body = _mlp_body if prev is None else _mlp_body_alias
  return pl.pallas_call(
      body,
      grid=grid,
      in_specs=ins,
      out_specs=[
          pl.BlockSpec((N_EFF, _BM), lambda i: (0, i + off)),
          pl.BlockSpec((N_OUT, _BM), lambda i: (0, i + off)),
      ],
      out_shape=[
          jax.ShapeDtypeStruct((N_EFF, B), jnp.float32),
          jax.ShapeDtypeStruct((N_OUT, B), jnp.float32),
      ],
      input_output_aliases=aliases,
  )(*args)


def _mlp_body_alias(xd_ref, xg_ref, w1d_ref, w1g_ref, b1_ref, w2_ref, b2_ref,
                    we_ref, be_ref, wo_ref, bo_ref, pe_ref, po_ref,
                    effT_ref, outT_ref):
  _mlp_body(xd_ref, xg_ref, w1d_ref, w1g_ref, b1_ref, w2_ref, b2_ref,
            we_ref, be_ref, wo_ref, bo_ref, effT_ref, outT_ref)


def kernel(drug, genotype, drug_emb, geno_emb, W1, b1, W2, b2, We, be, Wo, bo):
  dembq, gembq = _relayout(drug_emb.T, geno_emb.T)
  drug = drug.astype(jnp.int32)
  genotype = genotype.astype(jnp.int32)
  prev = None
  for c in range(_CHUNKS):
    sl = slice(c * _BC, (c + 1) * _BC)
    drug_f, geno_f = _sc_gather(drug[sl], genotype[sl], dembq, gembq)
    prev = _mlp(drug_f, geno_f,
                W1, b1, W2, b2, We, be, Wo, bo, c, prev)
  return (prev[0].T, prev[1].T)
